# baseline (speedup 1.0000x reference)
import functools

import jax
import jax.numpy as jnp
from jax import lax
from jax.experimental import pallas as pl
from jax.experimental.pallas import tpu as pltpu
from jax.experimental.pallas import tpu_sc as plsc

BATCH = 16384
DIM = 64
NENT = 1000000
NREL = 1000
HALF = NENT // 4

_GATHER_DNUMS = lax.GatherDimensionNumbers(
    offset_dims=(), collapsed_slice_dims=(0,), start_index_map=(0,)
)


def _vperm(v, p):
    return lax.gather(
        v,
        p.reshape(16, 1),
        _GATHER_DNUMS,
        (1,),
        mode=lax.GatherScatterMode.PROMISE_IN_BOUNDS,
    )


def _lane(v, r):
    splat = jnp.full((16,), r, jnp.int32)
    return lax.reduce_max(_vperm(v, splat), axes=(0,))


def _make_perms():
    lanes = lax.iota(jnp.int32, 16)
    fold = {k: (lanes + k // 2) & 15 for k in (16, 8, 4, 2)}
    pack = {k: (lanes - k // 2) & 15 for k in (16, 8, 4, 2)}
    odd = {
        k: ((lanes >> {8: 3, 4: 2, 2: 1, 1: 0}[k // 2]) & 1).astype(jnp.float32)
        for k in (16, 8, 4, 2)
    }
    bitrev = (
        ((lanes & 1) << 3) | ((lanes & 2) << 1) | ((lanes & 4) >> 1) | ((lanes & 8) >> 3)
    )
    return fold, pack, odd, bitrev


def _rowsum16(vecs, perms):
    fold, pack, odd, bitrev = perms

    def merge(a, b, k):
        a2 = a + _vperm(a, fold[k])
        b2 = b + _vperm(b, fold[k])
        pb = _vperm(b2, pack[k])
        return a2 + (pb - a2) * odd[k]

    k = 16
    while len(vecs) > 1:
        vecs = [merge(vecs[2 * i], vecs[2 * i + 1], k) for i in range(len(vecs) // 2)]
        k //= 2
    return _vperm(vecs[0], bitrev)


def kernel(e, l, t, Ee, El):
    info = plsc.get_sparse_core_info()
    nc, ns, nl = info.num_cores, info.num_subcores, info.num_lanes
    nw = nc * ns
    bpw = BATCH // nw
    chunk = 64
    nchunks = bpw // chunk

    ee_a = Ee[: NENT // 2].reshape(HALF, 2 * DIM)
    ee_b = Ee[NENT // 2 :].reshape(HALF, 2 * DIM)
    el2 = El.reshape(NREL // 2, 2 * DIM)

    mesh = plsc.VectorSubcoreMesh(core_axis_name="c", subcore_axis_name="s")

    @functools.partial(
        pl.kernel,
        mesh=mesh,
        out_type=jax.ShapeDtypeStruct((BATCH,), jnp.float32),
        compiler_params=pltpu.CompilerParams(needs_layout_passes=False),
        scratch_types=[
            pltpu.VMEM((bpw,), jnp.int32),
            pltpu.VMEM((bpw,), jnp.int32),
            pltpu.VMEM((bpw,), jnp.int32),
            pltpu.VMEM((bpw,), jnp.int32),
            pltpu.VMEM((bpw,), jnp.int32),
            pltpu.VMEM((bpw,), jnp.int32),
            pltpu.VMEM((bpw,), jnp.int32),
            pltpu.VMEM((bpw,), jnp.int32),
            pltpu.VMEM((chunk, 2 * DIM), jnp.float32),
            pltpu.VMEM((chunk, 2 * DIM), jnp.float32),
            pltpu.VMEM((chunk, 2 * DIM), jnp.float32),
            pltpu.VMEM((chunk, 2 * DIM), jnp.float32),
            pltpu.VMEM((chunk, 2 * DIM), jnp.float32),
            pltpu.VMEM((bpw,), jnp.float32),
            pltpu.SemaphoreType.DMA,
            pltpu.SemaphoreType.DMA,
            pltpu.SemaphoreType.DMA,
            pltpu.SemaphoreType.DMA,
            pltpu.SemaphoreType.DMA,
        ],
    )
    def trans_e(eea_hbm, eeb_hbm, el_hbm, e_hbm, l_hbm, t_hbm, out_hbm,
                ei_v, li_v, ti_v, ea_v, eb_v, ta_v, tb_v, lp_v,
                era_v, erb_v, tra_v, trb_v, lr_v, out_v,
                sem_ea, sem_eb, sem_ta, sem_tb, sem_l):
        wid = lax.axis_index("s") * nc + lax.axis_index("c")
        base = wid * bpw
        pltpu.sync_copy(e_hbm.at[pl.ds(base, bpw)], ei_v)
        pltpu.sync_copy(l_hbm.at[pl.ds(base, bpw)], li_v)
        pltpu.sync_copy(t_hbm.at[pl.ds(base, bpw)], ti_v)

        def pair_idx(i, carry):
            ds = pl.ds(i * nl, nl)
            pe = lax.shift_right_logical(ei_v[ds], 1)
            pt = lax.shift_right_logical(ti_v[ds], 1)
            ea_v[ds] = jnp.minimum(pe, HALF - 1)
            eb_v[ds] = jnp.maximum(pe - HALF, 0)
            ta_v[ds] = jnp.minimum(pt, HALF - 1)
            tb_v[ds] = jnp.maximum(pt - HALF, 0)
            lp_v[ds] = lax.shift_right_logical(li_v[ds], 1)
            return carry

        lax.fori_loop(0, bpw // nl, pair_idx, 0)

        def do_chunk(ck, carry):
            koff = ck * chunk
            kds = pl.ds(koff, chunk)
            c1 = pltpu.async_copy(eea_hbm.at[ea_v.at[kds]], era_v, sem_ea)
            c2 = pltpu.async_copy(eeb_hbm.at[eb_v.at[kds]], erb_v, sem_eb)
            c3 = pltpu.async_copy(eea_hbm.at[ta_v.at[kds]], tra_v, sem_ta)
            c4 = pltpu.async_copy(eeb_hbm.at[tb_v.at[kds]], trb_v, sem_tb)
            c5 = pltpu.async_copy(el_hbm.at[lp_v.at[kds]], lr_v, sem_l)
            c1.wait()
            c2.wait()
            c3.wait()
            c4.wait()
            c5.wait()

            def group(g, carry2):
                perms = _make_perms()
                g16 = g * nl
                iv_e = ei_v[pl.ds(koff + g16, nl)]
                iv_l = li_v[pl.ds(koff + g16, nl)]
                iv_t = ti_v[pl.ds(koff + g16, nl)]
                rows = []
                for r in range(nl):
                    row = g16 + r
                    se = _lane(iv_e, r)
                    st = _lane(iv_t, r)
                    sl = _lane(iv_l, r)
                    eo = (se & 1) * DIM
                    to = (st & 1) * DIM
                    lo = (sl & 1) * DIM
                    he = jnp.minimum(
                        lax.shift_right_logical(se, 1) // HALF, 1).astype(jnp.float32)
                    ht = jnp.minimum(
                        lax.shift_right_logical(st, 1) // HALF, 1).astype(jnp.float32)
                    acc = None
                    for c in range(DIM // nl):
                        dse = pl.ds(eo + c * nl, nl)
                        dst = pl.ds(to + c * nl, nl)
                        dsl = pl.ds(lo + c * nl, nl)
                        ev_a = era_v[row, dse]
                        ev = ev_a + (erb_v[row, dse] - ev_a) * he
                        tv_a = tra_v[row, dst]
                        tv = tv_a + (trb_v[row, dst] - tv_a) * ht
                        d = jnp.abs(ev + lr_v[row, dsl] - tv)
                        acc = d if acc is None else acc + d
                    rows.append(acc)
                out_v[pl.ds(koff + g16, nl)] = _rowsum16(rows, perms)
                return carry2

            lax.fori_loop(0, chunk // nl, group, 0)
            return carry

        lax.fori_loop(0, nchunks, do_chunk, 0)
        pltpu.sync_copy(out_v, out_hbm.at[pl.ds(base, bpw)])

    return trans_e(ee_a, ee_b, el2, e, l, t)


# speedup vs baseline: 4.0986x; 4.0986x over previous
import functools

import jax
import jax.numpy as jnp
from jax import lax
from jax.experimental import pallas as pl
from jax.experimental.pallas import tpu as pltpu
from jax.experimental.pallas import tpu_sc as plsc

BATCH = 16384
DIM = 64

_GATHER_DNUMS = lax.GatherDimensionNumbers(
    offset_dims=(), collapsed_slice_dims=(0,), start_index_map=(0,)
)


def _vperm(v, p):
    return lax.gather(
        v,
        p.reshape(16, 1),
        _GATHER_DNUMS,
        (1,),
        mode=lax.GatherScatterMode.PROMISE_IN_BOUNDS,
    )


def _lane(v, r):
    splat = jnp.full((16,), r, jnp.int32)
    return lax.reduce_max(_vperm(v, splat), axes=(0,))


def _make_perms():
    lanes = lax.iota(jnp.int32, 16)
    fold = {k: (lanes + k // 2) & 15 for k in (16, 8, 4, 2)}
    pack = {k: (lanes - k // 2) & 15 for k in (16, 8, 4, 2)}
    odd = {
        k: ((lanes >> {8: 3, 4: 2, 2: 1, 1: 0}[k // 2]) & 1).astype(jnp.float32)
        for k in (16, 8, 4, 2)
    }
    bitrev = (
        ((lanes & 1) << 3) | ((lanes & 2) << 1) | ((lanes & 4) >> 1) | ((lanes & 8) >> 3)
    )
    return fold, pack, odd, bitrev


def _rowsum16(vecs, perms):
    fold, pack, odd, bitrev = perms

    def merge(a, b, k):
        a2 = a + _vperm(a, fold[k])
        b2 = b + _vperm(b, fold[k])
        pb = _vperm(b2, pack[k])
        return a2 + (pb - a2) * odd[k]

    k = 16
    while len(vecs) > 1:
        vecs = [merge(vecs[2 * i], vecs[2 * i + 1], k) for i in range(len(vecs) // 2)]
        k //= 2
    return _vperm(vecs[0], bitrev)


def kernel(e, l, t, Ee, El):
    info = plsc.get_sparse_core_info()
    nc, ns, nl = info.num_cores, info.num_subcores, info.num_lanes
    nw = nc * ns
    bpw = BATCH // nw
    chunk = 256
    nchunks = bpw // chunk

    mesh = plsc.VectorSubcoreMesh(core_axis_name="c", subcore_axis_name="s")

    @functools.partial(
        pl.kernel,
        mesh=mesh,
        out_type=jax.ShapeDtypeStruct((BATCH,), jnp.float32),
        compiler_params=pltpu.CompilerParams(needs_layout_passes=False),
        scratch_types=[
            pltpu.VMEM((bpw,), jnp.int32),
            pltpu.VMEM((bpw,), jnp.int32),
            pltpu.VMEM((bpw,), jnp.int32),
            pltpu.VMEM((chunk, DIM), jnp.float32),
            pltpu.VMEM((chunk, DIM), jnp.float32),
            pltpu.VMEM((chunk, DIM), jnp.float32),
            pltpu.VMEM((bpw,), jnp.float32),
        ] + [pltpu.SemaphoreType.DMA] * 6,
    )
    def trans_e(ee_hbm, el_hbm, e_hbm, l_hbm, t_hbm, out_hbm,
                ei_v, li_v, ti_v, er_v, lr_v, tr_v, out_v,
                sem_e0, sem_e1, sem_l0, sem_l1, sem_t0, sem_t1):
        wid = lax.axis_index("s") * nc + lax.axis_index("c")
        base = wid * bpw
        pltpu.sync_copy(e_hbm.at[pl.ds(base, bpw)], ei_v)
        pltpu.sync_copy(l_hbm.at[pl.ds(base, bpw)], li_v)
        pltpu.sync_copy(t_hbm.at[pl.ds(base, bpw)], ti_v)

        def do_chunk(ck, carry):
            koff = ck * chunk

            def fire(g, carry2):
                g16 = g * nl
                iv_e = ei_v[pl.ds(koff + g16, nl)]
                iv_l = li_v[pl.ds(koff + g16, nl)]
                iv_t = ti_v[pl.ds(koff + g16, nl)]
                for r in range(nl):
                    row = g16 + r
                    se = (sem_e0, sem_e1)[r % 2]
                    sl = (sem_l0, sem_l1)[r % 2]
                    st = (sem_t0, sem_t1)[r % 2]
                    pltpu.async_copy(ee_hbm.at[_lane(iv_e, r)], er_v.at[row], se)
                    pltpu.async_copy(el_hbm.at[_lane(iv_l, r)], lr_v.at[row], sl)
                    pltpu.async_copy(ee_hbm.at[_lane(iv_t, r)], tr_v.at[row], st)
                return carry2

            lax.fori_loop(0, chunk // nl, fire, 0)

            h = chunk // 2
            for sem in (sem_e0, sem_e1, sem_l0, sem_l1, sem_t0, sem_t1):
                pltpu.make_async_copy(
                    ee_hbm.at[pl.ds(0, h)], er_v.at[pl.ds(0, h)], sem).wait()

            def group(g, carry2):
                perms = _make_perms()
                g16 = g * nl
                rows = []
                for r in range(nl):
                    row = g16 + r
                    acc = None
                    for c in range(DIM // nl):
                        ds = pl.ds(c * nl, nl)
                        d = jnp.abs(er_v[row, ds] + lr_v[row, ds] - tr_v[row, ds])
                        acc = d if acc is None else acc + d
                    rows.append(acc)
                out_v[pl.ds(koff + g16, nl)] = _rowsum16(rows, perms)
                return carry2

            lax.fori_loop(0, chunk // nl, group, 0)
            return carry

        lax.fori_loop(0, nchunks, do_chunk, 0)
        pltpu.sync_copy(out_v, out_hbm.at[pl.ds(base, bpw)])

    return trans_e(Ee, El, e, l, t)
